# baseline (speedup 1.0000x reference)
import jax
import jax.numpy as jnp
from jax import lax
from jax.experimental import pallas as pl
from jax.experimental.pallas import tpu as pltpu
from jax.experimental.pallas import tpu_sc as plsc

CLASSES = 1000
OUT_W = CLASSES + 1
B, L = 4096, 20
N = B * L

NUM_CORES = 2
NUM_SUBCORES = 16
NW = NUM_CORES * NUM_SUBCORES
ROWS_PER_W = N // NW
R = 32
NCH = ROWS_PER_W // R
G = R // 16


def _sc_body(acts_hbm, durs_hbm, out_hbm, acts_v, durs_v, buf0, buf1, sem0, sem1):
    wid = lax.axis_index("s") * NUM_CORES + lax.axis_index("c")
    base0 = wid * ROWS_PER_W

    pltpu.sync_copy(acts_hbm.at[pl.ds(base0, ROWS_PER_W)], acts_v)
    pltpu.sync_copy(durs_hbm.at[pl.ds(base0, ROWS_PER_W)], durs_v)

    zeros16 = jnp.zeros((16,), jnp.float32)
    ones16 = jnp.ones((16,), jnp.float32)
    col_last = jnp.full((16,), CLASSES, jnp.int32)
    iota16 = lax.iota(jnp.int32, 16)

    def zero_row(r, _):
        for buf in (buf0, buf1):
            for j in range(62):
                buf[r, pl.ds(j * 16, 16)] = zeros16
            buf[r, pl.ds(OUT_W - 16, 16)] = zeros16
        return 0

    lax.fori_loop(0, R, zero_row, 0)

    def fill(buf, sem, chunk):
        off = chunk * R
        for g in range(G):
            a16 = acts_v[pl.ds(off + g * 16, 16)]
            d16 = durs_v[pl.ds(off + g * 16, 16)]
            rows = iota16 + g * 16
            plsc.store_scatter(buf, [rows, a16], ones16)
            plsc.store_scatter(buf, [rows, col_last], d16)
        pltpu.make_async_copy(buf, out_hbm.at[pl.ds(base0 + off, R)], sem).start()

    def clean(buf, sem, chunk):
        off = chunk * R
        pltpu.make_async_copy(buf, out_hbm.at[pl.ds(base0 + off, R)], sem).wait()
        for g in range(G):
            a16 = acts_v[pl.ds(off + g * 16, 16)]
            rows = iota16 + g * 16
            plsc.store_scatter(buf, [rows, a16], zeros16)

    fill(buf0, sem0, 0)
    fill(buf1, sem1, 1)

    def body(p, _):
        c0 = 2 * p
        clean(buf0, sem0, c0 - 2)
        fill(buf0, sem0, c0)
        clean(buf1, sem1, c0 - 1)
        fill(buf1, sem1, c0 + 1)
        return 0

    lax.fori_loop(1, NCH // 2, body, 0)

    pltpu.make_async_copy(
        buf0, out_hbm.at[pl.ds(base0 + (NCH - 2) * R, R)], sem0
    ).wait()
    pltpu.make_async_copy(
        buf1, out_hbm.at[pl.ds(base0 + (NCH - 1) * R, R)], sem1
    ).wait()


_sc_call = pl.kernel(
    _sc_body,
    out_type=jax.ShapeDtypeStruct((N, OUT_W), jnp.float32),
    mesh=plsc.VectorSubcoreMesh(core_axis_name="c", subcore_axis_name="s"),
    scratch_types=[
        pltpu.VMEM((ROWS_PER_W,), jnp.int32),
        pltpu.VMEM((ROWS_PER_W,), jnp.float32),
        pltpu.VMEM((R, OUT_W), jnp.float32),
        pltpu.VMEM((R, OUT_W), jnp.float32),
        pltpu.SemaphoreType.DMA,
        pltpu.SemaphoreType.DMA,
    ],
    compiler_params=pltpu.CompilerParams(
        use_tc_tiling_on_sc=False, needs_layout_passes=False
    ),
)


def kernel(x):
    acts = x[..., 0].astype(jnp.int32).reshape(N)
    durs = x[..., 1].reshape(N)
    out = _sc_call(acts, durs)
    return out.reshape(B, L, OUT_W)
